# baseline (speedup 1.0000x reference)
import jax
import jax.numpy as jnp
from jax import lax
from jax.experimental import pallas as pl
from jax.experimental.pallas import tpu as pltpu
from jax.experimental.pallas import tpu_sc as plsc

_N = 1024
_K = 512
_D = 256
_B = 256
_NB = _N // _B


def _dist_argmin_body(z_ref, e_ref, idx_ref, loss_ref):
    i = pl.program_id(0)
    z = z_ref[:]
    e = e_ref[:]
    g = lax.dot_general(
        z, e, (((1,), (1,)), ((), ())),
        precision=lax.Precision.HIGHEST,
        preferred_element_type=jnp.float32)
    en_row = lax.dot_general(
        jnp.ones((1, _D), jnp.float32), e * e, (((1,), (1,)), ((), ())),
        precision=lax.Precision.HIGHEST,
        preferred_element_type=jnp.float32)
    m = en_row - 2.0 * g
    mmin = jnp.min(m, axis=1, keepdims=True)
    iota = lax.broadcasted_iota(jnp.int32, (_B, _K), 1)
    idx = jnp.min(jnp.where(m <= mmin, iota, _K), axis=1, keepdims=True)
    zn = jnp.sum(z * z, axis=1, keepdims=True)
    idx_ref[...] = idx.reshape(_B)
    part = jnp.sum(zn + mmin) * (1.0 / (_N * _D))

    @pl.when(i == 0)
    def _init():
        loss_ref[0, 0] = part

    @pl.when(i > 0)
    def _acc():
        loss_ref[0, 0] += part


_dist_call = pl.pallas_call(
    _dist_argmin_body,
    grid=(_NB,),
    in_specs=(
        pl.BlockSpec((_B, _D), lambda i: (i, 0)),
        pl.BlockSpec((_K, _D), lambda i: (0, 0)),
    ),
    out_shape=(
        jax.ShapeDtypeStruct((_N,), jnp.int32),
        jax.ShapeDtypeStruct((1, 1), jnp.float32),
    ),
    out_specs=(
        pl.BlockSpec((_B,), lambda i: (i,)),
        pl.BlockSpec(memory_space=pltpu.SMEM),
    ),
)


_NC = 2
_NS = 16
_NW = _NC * _NS
_BPW = _N // _NW
_CH = 8
_NCH = _BPW // _CH


def _gather_body(table_hbm, idx_hbm, out_hbm, idx_v, rows_v, gsem, ssem):
    wid = lax.axis_index("s") * _NC + lax.axis_index("c")
    base = wid * _BPW
    pltpu.sync_copy(idx_hbm.at[pl.ds(base, _BPW)], idx_v)
    gets = [
        pltpu.async_copy(
            table_hbm.at[idx_v.at[pl.ds(c * _CH, _CH)]],
            rows_v.at[pl.ds(c * _CH, _CH)], gsem)
        for c in range(_NCH)
    ]
    puts = []
    for c in range(_NCH):
        gets[c].wait()
        puts.append(pltpu.async_copy(
            rows_v.at[pl.ds(c * _CH, _CH)],
            out_hbm.at[pl.ds(base + c * _CH, _CH)], ssem))
    for p in puts:
        p.wait()


_gather = pl.kernel(
    _gather_body,
    out_type=jax.ShapeDtypeStruct((_N, _D), jnp.float32),
    mesh=plsc.VectorSubcoreMesh(core_axis_name="c", subcore_axis_name="s"),
    scratch_types=[
        pltpu.VMEM((_BPW,), jnp.int32),
        pltpu.VMEM((_BPW, _D), jnp.float32),
        pltpu.SemaphoreType.DMA,
        pltpu.SemaphoreType.DMA,
    ],
)


def kernel(z, embeddings):
    idx, loss2 = _dist_call(z, embeddings)
    z_q = _gather(embeddings, idx)
    return (z_q, loss2[0, 0], idx)


# speedup vs baseline: 1.0608x; 1.0608x over previous
import jax
import jax.numpy as jnp
from jax import lax
from jax.experimental import pallas as pl
from jax.experimental.pallas import tpu as pltpu
from jax.experimental.pallas import tpu_sc as plsc

_N = 1024
_K = 512
_D = 256
_H = _N // 2


def _dist_argmin_body(z_hbm, e_ref, idx_ref, loss_ref, z_v, sem0, sem1):
    cp0 = pltpu.make_async_copy(z_hbm.at[pl.ds(0, _H)], z_v.at[pl.ds(0, _H)], sem0)
    cp1 = pltpu.make_async_copy(z_hbm.at[pl.ds(_H, _H)], z_v.at[pl.ds(_H, _H)], sem1)
    cp0.start()
    cp1.start()
    e = e_ref[:]
    en_row = lax.dot_general(
        jnp.ones((1, _D), jnp.float32), e * e, (((1,), (1,)), ((), ())),
        precision=lax.Precision.HIGHEST,
        preferred_element_type=jnp.float32)

    def _half(z, base):
        g = lax.dot_general(
            z, e, (((1,), (1,)), ((), ())),
            precision=lax.Precision.HIGHEST,
            preferred_element_type=jnp.float32)
        m = en_row - 2.0 * g
        mmin = jnp.min(m, axis=1, keepdims=True)
        iota = lax.broadcasted_iota(jnp.int32, (_H, _K), 1)
        idx = jnp.min(jnp.where(m <= mmin, iota, _K), axis=1, keepdims=True)
        zn = jnp.sum(z * z, axis=1, keepdims=True)
        idx_ref[pl.ds(base, _H)] = idx.reshape(_H)
        return jnp.sum(zn + mmin) * (1.0 / (_N * _D))

    cp0.wait()
    p0 = _half(z_v[pl.ds(0, _H), :], 0)
    cp1.wait()
    p1 = _half(z_v[pl.ds(_H, _H), :], _H)
    loss_ref[0, 0] = p0 + p1


_dist_call = pl.pallas_call(
    _dist_argmin_body,
    in_specs=(
        pl.BlockSpec(memory_space=pl.ANY),
        pl.BlockSpec(memory_space=pltpu.VMEM),
    ),
    out_shape=(
        jax.ShapeDtypeStruct((_N,), jnp.int32),
        jax.ShapeDtypeStruct((1, 1), jnp.float32),
    ),
    out_specs=(
        pl.BlockSpec(memory_space=pltpu.VMEM),
        pl.BlockSpec(memory_space=pltpu.SMEM),
    ),
    scratch_shapes=[
        pltpu.VMEM((_N, _D), jnp.float32),
        pltpu.SemaphoreType.DMA,
        pltpu.SemaphoreType.DMA,
    ],
)


_NC = 2
_NS = 16
_NW = _NC * _NS
_BPW = _N // _NW


def _gather_body(table_hbm, idx_hbm, out_hbm, idx_v, rows_v, sem):
    wid = lax.axis_index("s") * _NC + lax.axis_index("c")
    base = wid * _BPW
    pltpu.sync_copy(idx_hbm.at[pl.ds(base, _BPW)], idx_v)
    pltpu.async_copy(table_hbm.at[idx_v], rows_v, sem).wait()
    pltpu.sync_copy(rows_v, out_hbm.at[pl.ds(base, _BPW)])


_gather = pl.kernel(
    _gather_body,
    out_type=jax.ShapeDtypeStruct((_N, _D), jnp.float32),
    mesh=plsc.VectorSubcoreMesh(core_axis_name="c", subcore_axis_name="s"),
    scratch_types=[
        pltpu.VMEM((_BPW,), jnp.int32),
        pltpu.VMEM((_BPW, _D), jnp.float32),
        pltpu.SemaphoreType.DMA,
    ],
)


def kernel(z, embeddings):
    idx, loss2 = _dist_call(z, embeddings)
    z_q = _gather(embeddings, idx)
    return (z_q, loss2[0, 0], idx)


# speedup vs baseline: 1.0608x; 1.0000x over previous
import jax
import jax.numpy as jnp
from jax import lax
from jax.experimental import pallas as pl
from jax.experimental.pallas import tpu as pltpu
from jax.experimental.pallas import tpu_sc as plsc

_N = 1024
_K = 512
_D = 256


_B = 512
_NB = _N // _B


def _dist_argmin_body(z_ref, e_ref, idx_ref, loss_ref):
    i = pl.program_id(0)
    z = z_ref[:]
    e = e_ref[:]
    g = lax.dot_general(
        z, e, (((1,), (1,)), ((), ())),
        precision=lax.Precision.HIGHEST,
        preferred_element_type=jnp.float32)
    en_row = lax.dot_general(
        jnp.ones((1, _D), jnp.float32), e * e, (((1,), (1,)), ((), ())),
        precision=lax.Precision.HIGHEST,
        preferred_element_type=jnp.float32)
    m = en_row - 2.0 * g
    mmin = jnp.min(m, axis=1, keepdims=True)
    iota = lax.broadcasted_iota(jnp.int32, (_B, _K), 1)
    idx = jnp.min(jnp.where(m <= mmin, iota, _K), axis=1, keepdims=True)
    zn = jnp.sum(z * z, axis=1, keepdims=True)
    idx_ref[...] = idx.reshape(_B)
    part = jnp.sum(zn + mmin) * (1.0 / (_N * _D))

    @pl.when(i == 0)
    def _init():
        loss_ref[0, 0] = part

    @pl.when(i > 0)
    def _acc():
        loss_ref[0, 0] += part


_dist_call = pl.pallas_call(
    _dist_argmin_body,
    grid=(_NB,),
    in_specs=(
        pl.BlockSpec((_B, _D), lambda i: (i, 0)),
        pl.BlockSpec((_K, _D), lambda i: (0, 0)),
    ),
    out_shape=(
        jax.ShapeDtypeStruct((_N,), jnp.int32),
        jax.ShapeDtypeStruct((1, 1), jnp.float32),
    ),
    out_specs=(
        pl.BlockSpec((_B,), lambda i: (i,)),
        pl.BlockSpec(memory_space=pltpu.SMEM),
    ),
)


_NC = 2
_NS = 16
_NW = _NC * _NS
_BPW = _N // _NW


_CH = _BPW // 2


def _gather_body(table_hbm, idx_hbm, out_hbm, idx_v, rows_v, gsem, ssem):
    wid = lax.axis_index("s") * _NC + lax.axis_index("c")
    base = wid * _BPW
    pltpu.sync_copy(idx_hbm.at[pl.ds(base, _BPW)], idx_v)
    g0 = pltpu.async_copy(table_hbm.at[idx_v.at[pl.ds(0, _CH)]],
                          rows_v.at[pl.ds(0, _CH)], gsem)
    g1 = pltpu.async_copy(table_hbm.at[idx_v.at[pl.ds(_CH, _CH)]],
                          rows_v.at[pl.ds(_CH, _CH)], gsem)
    g0.wait()
    s0 = pltpu.async_copy(rows_v.at[pl.ds(0, _CH)],
                          out_hbm.at[pl.ds(base, _CH)], ssem)
    g1.wait()
    s1 = pltpu.async_copy(rows_v.at[pl.ds(_CH, _CH)],
                          out_hbm.at[pl.ds(base + _CH, _CH)], ssem)
    s0.wait()
    s1.wait()


_gather = pl.kernel(
    _gather_body,
    out_type=jax.ShapeDtypeStruct((_N, _D), jnp.float32),
    mesh=plsc.VectorSubcoreMesh(core_axis_name="c", subcore_axis_name="s"),
    scratch_types=[
        pltpu.VMEM((_BPW,), jnp.int32),
        pltpu.VMEM((_BPW, _D), jnp.float32),
        pltpu.SemaphoreType.DMA,
        pltpu.SemaphoreType.DMA,
    ],
)


def kernel(z, embeddings):
    idx, loss2 = _dist_call(z, embeddings)
    z_q = _gather(embeddings, idx)
    return (z_q, loss2[0, 0], idx)


# speedup vs baseline: 1.1027x; 1.0395x over previous
import jax
import jax.numpy as jnp
from jax import lax
from jax.experimental import pallas as pl
from jax.experimental.pallas import tpu as pltpu
from jax.experimental.pallas import tpu_sc as plsc

_N = 1024
_K = 512
_D = 256


def _dist_argmin_body(z_ref, e_ref, idx_ref, loss_ref):
    z = z_ref[:]
    e = e_ref[:]
    g = lax.dot_general(
        z, e, (((1,), (1,)), ((), ())),
        precision=lax.Precision.HIGHEST,
        preferred_element_type=jnp.float32)
    en_row = lax.dot_general(
        jnp.ones((1, _D), jnp.float32), e * e, (((1,), (1,)), ((), ())),
        precision=lax.Precision.HIGHEST,
        preferred_element_type=jnp.float32)
    m = en_row - 2.0 * g
    mmin = jnp.min(m, axis=1, keepdims=True)
    iota = lax.broadcasted_iota(jnp.int32, (_N, _K), 1)
    idx = jnp.min(jnp.where(m <= mmin, iota, _K), axis=1, keepdims=True)
    zn = jnp.sum(z * z, axis=1, keepdims=True)
    idx_ref[...] = idx.reshape(_N)
    loss_ref[0, 0] = jnp.sum(zn + mmin) / (_N * _D)


_dist_call = pl.pallas_call(
    _dist_argmin_body,
    out_shape=(
        jax.ShapeDtypeStruct((_N,), jnp.int32),
        jax.ShapeDtypeStruct((1, 1), jnp.float32),
    ),
    out_specs=(
        pl.BlockSpec(memory_space=pltpu.VMEM),
        pl.BlockSpec(memory_space=pltpu.SMEM),
    ),
)


_NC = 2
_NS = 16
_NW = _NC * _NS
_BPW = _N // _NW


def _gather_body(table_hbm, idx_hbm, out_hbm, idx_v, rows_v, sem):
    wid = lax.axis_index("s") * _NC + lax.axis_index("c")
    base = wid * _BPW
    pltpu.sync_copy(idx_hbm.at[pl.ds(base, _BPW)], idx_v)
    pltpu.async_copy(table_hbm.at[idx_v], rows_v, sem).wait()
    pltpu.sync_copy(rows_v, out_hbm.at[pl.ds(base, _BPW)])


_gather = pl.kernel(
    _gather_body,
    out_type=jax.ShapeDtypeStruct((_N, _D), jnp.float32),
    mesh=plsc.VectorSubcoreMesh(core_axis_name="c", subcore_axis_name="s"),
    scratch_types=[
        pltpu.VMEM((_BPW,), jnp.int32),
        pltpu.VMEM((_BPW, _D), jnp.float32),
        pltpu.SemaphoreType.DMA,
    ],
)


def kernel(z, embeddings):
    idx, loss2 = _dist_call(z, embeddings)
    z_q = _gather(embeddings, idx)
    return (z_q, loss2[0, 0], idx)


# speedup vs baseline: 1.1523x; 1.0450x over previous
import jax
import jax.numpy as jnp
from jax import lax
from jax.experimental import pallas as pl
from jax.experimental.pallas import tpu as pltpu
from jax.experimental.pallas import tpu_sc as plsc

_N = 1024
_K = 512
_D = 256


def _dist_argmin_body(z_ref, e_ref, idx_ref, loss_ref):
    z = z_ref[:]
    e = e_ref[:]
    gt = lax.dot_general(
        e, z, (((1,), (1,)), ((), ())),
        precision=lax.Precision.HIGHEST,
        preferred_element_type=jnp.float32)
    en_col = jnp.sum(e * e, axis=1, keepdims=True)
    mt = en_col - 2.0 * gt
    mmin = jnp.min(mt, axis=0, keepdims=True)
    iota = lax.broadcasted_iota(jnp.int32, (_K, _N), 0)
    idx = jnp.min(jnp.where(mt <= mmin, iota, _K), axis=0, keepdims=True)
    idx_ref[...] = idx.reshape(_N)
    loss_ref[0, 0] = (jnp.sum(z * z) + jnp.sum(mmin)) / (_N * _D)


_dist_call = pl.pallas_call(
    _dist_argmin_body,
    out_shape=(
        jax.ShapeDtypeStruct((_N,), jnp.int32),
        jax.ShapeDtypeStruct((1, 1), jnp.float32),
    ),
    out_specs=(
        pl.BlockSpec(memory_space=pltpu.VMEM),
        pl.BlockSpec(memory_space=pltpu.SMEM),
    ),
)


_NC = 2
_NS = 16
_NW = _NC * _NS
_BPW = _N // _NW


def _gather_body(table_hbm, idx_hbm, out_hbm, idx_v, rows_v, sem):
    wid = lax.axis_index("s") * _NC + lax.axis_index("c")
    base = wid * _BPW
    pltpu.sync_copy(idx_hbm.at[pl.ds(base, _BPW)], idx_v)
    pltpu.async_copy(table_hbm.at[idx_v], rows_v, sem).wait()
    pltpu.sync_copy(rows_v, out_hbm.at[pl.ds(base, _BPW)])


_gather = pl.kernel(
    _gather_body,
    out_type=jax.ShapeDtypeStruct((_N, _D), jnp.float32),
    mesh=plsc.VectorSubcoreMesh(core_axis_name="c", subcore_axis_name="s"),
    scratch_types=[
        pltpu.VMEM((_BPW,), jnp.int32),
        pltpu.VMEM((_BPW, _D), jnp.float32),
        pltpu.SemaphoreType.DMA,
    ],
)


def kernel(z, embeddings):
    idx, loss2 = _dist_call(z, embeddings)
    z_q = _gather(embeddings, idx)
    return (z_q, loss2[0, 0], idx)
